# baseline (speedup 1.0000x reference)
import functools

import jax
import jax.numpy as jnp
from jax.experimental import pallas as pl
from jax.experimental.pallas import tpu as pltpu

_INTERP = False

N = 512
D = 68
K = 8
EO = 32
EMB = 32
NCLS = 2
BB = 2
L = N * (N - 1) // 2
NBLK = N // 8


def _leaky(x):
    return jnp.maximum(x, 0.01 * x)


def _mm(a, b):
    return jax.lax.dot_general(a, b, (((1,), (0,)), ((), ())),
                               preferred_element_type=jnp.float32)


def _body(feat_r, bnw_r, bnb_r,
          e1aT_r, e1ab_r, e1bT_r, e1bb_r,
          e2aT_r, e2ab_r, e2bT_r, e2bb_r,
          w1lf_r, w1le1_r, w1le2_r, w1rf_r, w1re1_r, w1re2_r,
          bwlf_r, bwle1_r, bwle2_r, bwrf_r, bwre1_r, bwre2_r,
          bblf_r, bble1_r, bble2_r, bbrf_r, bbre1_r, bbre2_r,
          l1b_r, l2w_r, l2b_r, l3w_r, l3b_r,
          out_r, ocat_r, at_r, at1_r):
    f_all = feat_r[...]
    m1 = jnp.mean(f_all, axis=0, keepdims=True)
    xc = f_all - m1
    v1 = jnp.mean(xc * xc, axis=0, keepdims=True)
    fn_all = xc * (bnw_r[...] / jnp.sqrt(v1 + 1e-5)) + bnb_r[...]

    col_i = jax.lax.broadcasted_iota(jnp.int32, (N, N), 1)
    segs = []
    for b in range(BB):
        fnb = fn_all[b * N:(b + 1) * N, :]
        pos_t = jnp.transpose(fnb[:, 0:4])
        dist = jnp.abs(fnb[:, 0:1] - pos_t[0:1, :])
        for c in range(1, 4):
            dist = dist + jnp.abs(fnb[:, c:c + 1] - pos_t[c:c + 1, :])
        for k in range(K):
            mn = jnp.min(dist, axis=1, keepdims=True)
            cand = jnp.where(dist == mn, col_i, N)
            am = jnp.min(cand, axis=1, keepdims=True)
            hit = col_i == am
            ocat_r[:, k * N:(k + 1) * N] = hit.astype(jnp.float32)
            dist = jnp.where(hit, 1e30, dist)
        oc = ocat_r[...]

        pstack = jnp.concatenate(
            [_mm(fnb, e1aT_r[k * D:(k + 1) * D, :]) for k in range(K)], axis=0)
        e1 = _leaky(_mm(oc, pstack) + e1ab_r[...])
        e1 = _leaky(_mm(e1, e1bT_r[...]) + e1bb_r[...])
        qstack = jnp.concatenate(
            [_mm(e1, e2aT_r[k * EO:(k + 1) * EO, :]) for k in range(K)], axis=0)
        e2 = _leaky(_mm(oc, qstack) + e2ab_r[...])
        e2 = _leaky(_mm(e2, e2bT_r[...]) + e2bb_r[...])
        segs.append((fnb, e1, e2))

    riota = jax.lax.broadcasted_iota(jnp.int32, (N, 1), 0).astype(jnp.float32)
    wl = (N - 1.0) - riota
    wr = riota
    denom = 1.0 / (BB * L)

    def side_act(w_row, wsegs, bwsegs, bbsegs):
        acts = []
        for b in range(BB):
            acc = None
            for s in range(3):
                S0 = segs[0][s]
                S1 = segs[1][s]
                sm = (jnp.sum(w_row * S0, axis=0, keepdims=True)
                      + jnp.sum(w_row * S1, axis=0, keepdims=True)) * denom
                d0 = S0 - sm
                d1 = S1 - sm
                sv = (jnp.sum(w_row * d0 * d0, axis=0, keepdims=True)
                      + jnp.sum(w_row * d1 * d1, axis=0, keepdims=True)) * denom
                sc = bwsegs[s][...] / jnp.sqrt(sv + 1e-5)
                sh = bbsegs[s][...] - sm * sc
                t = _mm(segs[b][s] * sc + sh, wsegs[s][...])
                acc = t if acc is None else acc + t
            acts.append(acc)
        return acts

    A01 = side_act(wl, (w1lf_r, w1le1_r, w1le2_r),
                   (bwlf_r, bwle1_r, bwle2_r), (bblf_r, bble1_r, bble2_r))
    B01 = side_act(wr, (w1rf_r, w1re1_r, w1re2_r),
                   (bwrf_r, bwre1_r, bwre2_r), (bbrf_r, bbre1_r, bbre2_r))

    l1b = l1b_r[...]
    l2w = l2w_r[...]
    l2b = l2b_r[...]
    l3w = l3w_r[...]
    l3b = l3b_r[...]

    ri2 = jax.lax.broadcasted_iota(jnp.int32, (N, N), 0)
    revm = (ri2 + col_i == N - 1).astype(jnp.float32)
    rowU = jnp.transpose(A01[0] + l1b)
    rowL = _mm(jnp.transpose(A01[1] + l1b), revm)
    colU = jnp.transpose(B01[0])
    colL = _mm(jnp.transpose(B01[1]), revm)
    for ib in range(NBLK):
        at_r[ib] = rowU[:, ib * 8:(ib + 1) * 8]
        at1_r[ib] = rowL[:, ib * 8:(ib + 1) * 8]

    def store_win(row_sel, base, lo, hi, vec, lane, zpad):
        sh = jax.lax.rem(base, 128)
        b128 = pl.multiple_of(base - sh, 128)
        rolled = pltpu.roll(jnp.concatenate([vec, zpad], axis=1), sh, axis=1)
        keep = (lane >= sh + lo) & (lane < sh + hi)
        win = out_r[row_sel, pl.ds(b128, N + 128)]
        out_r[row_sel, pl.ds(b128, N + 128)] = jnp.where(keep, rolled, win)

    def pbody(ib, carry):
        a0 = at_r[ib]
        a1 = at1_r[ib]
        ri = jax.lax.broadcasted_iota(jnp.int32, (8, N), 0) + ib * 8
        cj = jax.lax.broadcasted_iota(jnp.int32, (8, N), 1)
        m = (cj > ri)[None, :, :]
        h = (jnp.where(m, a0[:, :, None], a1[:, :, None])
             + jnp.where(m, colU[:, None, :], colL[:, None, :]))
        h = _leaky(h).reshape(EMB, 8 * N)
        h2 = _leaky(_mm(l2w, h) + l2b)
        o = _mm(l3w, h2) + l3b
        lane = jax.lax.broadcasted_iota(jnp.int32, (2, N + 128), 1)
        zpad = jnp.zeros((2, 128), jnp.float32)
        for r in range(8):
            i = ib * 8 + r
            i1 = (N - 1) - i
            off0 = i * (N - 1) - (i * (i - 1)) // 2
            off1 = i1 * (N - 1) - (i1 * (i1 - 1)) // 2
            vec0 = o[:, r * N:(r + 1) * N]
            store_win(slice(0, 2), off0 - i, i + 1, N, vec0, lane, zpad)
            store_win(slice(2, 4), L - off1 - i + 1, 0, i, vec0, lane, zpad)
        return carry

    jax.lax.fori_loop(0, NBLK, pbody, 0)


@jax.jit
def kernel(feat, bn_w, bn_b, bn2_w, bn2_b, l1_W, l1_b, l2_W, l2_b, l3_W, l3_b,
           e1a_W, e1a_b, e1b_W, e1b_b, e2a_W, e2a_b, e2b_W, e2b_b):
    feat2d = feat.reshape(BB * N, D)
    row = lambda v: v.reshape(1, -1)
    colv = lambda v: v.reshape(-1, 1)
    e1aT = e1a_W.reshape(EO, D, K).transpose(2, 1, 0).reshape(K * D, EO)
    e2aT = e2a_W.reshape(EO, EO, K).transpose(2, 1, 0).reshape(K * EO, EO)
    F = D + 2 * EO
    W1L, W1R = l1_W[:, :F], l1_W[:, F:]
    seg3 = lambda M: (M[:, :D].T, M[:, D:D + EO].T, M[:, D + EO:].T)
    segv = lambda v: (row(v[:D]), row(v[D:D + EO]), row(v[D + EO:]))
    w1l = seg3(W1L)
    w1r = seg3(W1R)
    bwl = segv(bn2_w[:F])
    bwr = segv(bn2_w[F:])
    bbl = segv(bn2_b[:F])
    bbr = segv(bn2_b[F:])

    operands = (feat2d, row(bn_w), row(bn_b),
                e1aT, row(e1a_b), e1b_W.T, row(e1b_b),
                e2aT, row(e2a_b), e2b_W.T, row(e2b_b),
                *w1l, *w1r, *bwl, *bwr, *bbl, *bbr,
                row(l1_b), l2_W, colv(l2_b), l3_W, colv(l3_b))

    out = pl.pallas_call(
        _body,
        out_shape=jax.ShapeDtypeStruct((2 * BB, 131072), jnp.float32),
        scratch_shapes=[pltpu.VMEM((N, K * N), jnp.float32),
                        pltpu.VMEM((NBLK, EMB, 8), jnp.float32),
                        pltpu.VMEM((NBLK, EMB, 8), jnp.float32)],
        interpret=_INTERP,
    )(*operands)

    p0 = out[0:2, 1:L + 1]
    p1 = jnp.flip(out[2:4, 1:L + 1], axis=1)
    preds = jnp.stack([p0, p1]).transpose(0, 2, 1)
    cells = feat[:, :, :4]
    return preds, cells


# speedup vs baseline: 2.0658x; 2.0658x over previous
import functools

import jax
import jax.numpy as jnp
from jax.experimental import pallas as pl
from jax.experimental.pallas import tpu as pltpu

_INTERP = False

N = 512
D = 68
K = 8
EO = 32
EMB = 32
NCLS = 2
BB = 2
L = N * (N - 1) // 2
NBLK = N // 8


def _leaky(x):
    return jnp.maximum(x, 0.01 * x)


def _mm(a, b):
    return jax.lax.dot_general(a, b, (((1,), (0,)), ((), ())),
                               preferred_element_type=jnp.float32)


def _body(feat_r, bnw_r, bnb_r,
          e1aT_r, e1ab_r, e1bT_r, e1bb_r,
          e2aT_r, e2ab_r, e2bT_r, e2bb_r,
          w1lf_r, w1le1_r, w1le2_r, w1rf_r, w1re1_r, w1re2_r,
          bwlf_r, bwle1_r, bwle2_r, bwrf_r, bwre1_r, bwre2_r,
          bblf_r, bble1_r, bble2_r, bbrf_r, bbre1_r, bbre2_r,
          l1b_r, l2w_r, l2b_r, l3w_r, l3b_r,
          out_r, ocat_r, at_r):
    f_all = feat_r[...]
    m1 = jnp.mean(f_all, axis=0, keepdims=True)
    xc = f_all - m1
    v1 = jnp.mean(xc * xc, axis=0, keepdims=True)
    fn_all = xc * (bnw_r[...] / jnp.sqrt(v1 + 1e-5)) + bnb_r[...]

    col_i = jax.lax.broadcasted_iota(jnp.int32, (N, N), 1)
    segs = []
    for b in range(BB):
        fnb = fn_all[b * N:(b + 1) * N, :]
        pos_t = jnp.transpose(fnb[:, 0:4])
        dist = jnp.abs(fnb[:, 0:1] - pos_t[0:1, :])
        for c in range(1, 4):
            dist = dist + jnp.abs(fnb[:, c:c + 1] - pos_t[c:c + 1, :])
        for k in range(K):
            mn = jnp.min(dist, axis=1, keepdims=True)
            cand = jnp.where(dist == mn, col_i, N)
            am = jnp.min(cand, axis=1, keepdims=True)
            hit = col_i == am
            ocat_r[:, k * N:(k + 1) * N] = hit.astype(jnp.float32)
            dist = jnp.where(hit, 1e30, dist)
        oc = ocat_r[...]

        pstack = jnp.concatenate(
            [_mm(fnb, e1aT_r[k * D:(k + 1) * D, :]) for k in range(K)], axis=0)
        e1 = _leaky(_mm(oc, pstack) + e1ab_r[...])
        e1 = _leaky(_mm(e1, e1bT_r[...]) + e1bb_r[...])
        qstack = jnp.concatenate(
            [_mm(e1, e2aT_r[k * EO:(k + 1) * EO, :]) for k in range(K)], axis=0)
        e2 = _leaky(_mm(oc, qstack) + e2ab_r[...])
        e2 = _leaky(_mm(e2, e2bT_r[...]) + e2bb_r[...])
        segs.append((fnb, e1, e2))

    riota = jax.lax.broadcasted_iota(jnp.int32, (N, 1), 0).astype(jnp.float32)
    wl = (N - 1.0) - riota
    wr = riota
    denom = 1.0 / (BB * L)

    def side_act(w_row, wsegs, bwsegs, bbsegs):
        acts = []
        for b in range(BB):
            acc = None
            for s in range(3):
                S0 = segs[0][s]
                S1 = segs[1][s]
                sm = (jnp.sum(w_row * S0, axis=0, keepdims=True)
                      + jnp.sum(w_row * S1, axis=0, keepdims=True)) * denom
                d0 = S0 - sm
                d1 = S1 - sm
                sv = (jnp.sum(w_row * d0 * d0, axis=0, keepdims=True)
                      + jnp.sum(w_row * d1 * d1, axis=0, keepdims=True)) * denom
                sc = bwsegs[s][...] / jnp.sqrt(sv + 1e-5)
                sh = bbsegs[s][...] - sm * sc
                t = _mm(segs[b][s] * sc + sh, wsegs[s][...])
                acc = t if acc is None else acc + t
            acts.append(acc)
        return acts

    A01 = side_act(wl, (w1lf_r, w1le1_r, w1le2_r),
                   (bwlf_r, bwle1_r, bwle2_r), (bblf_r, bble1_r, bble2_r))
    B01 = side_act(wr, (w1rf_r, w1re1_r, w1re2_r),
                   (bwrf_r, bwre1_r, bwre2_r), (bbrf_r, bbre1_r, bbre2_r))

    l1b = l1b_r[...]
    l2w = l2w_r[...]
    l2b = l2b_r[...]
    l3w = l3w_r[...]
    l3b = l3b_r[...]

    zpad = jnp.zeros((2, 128), jnp.float32)

    def store_win(row_sel, base, lo, vec, wq, lane):
        sh = jax.lax.rem(base, 128)
        b128 = pl.multiple_of(base - sh, 128)
        rolled = pltpu.roll(jnp.concatenate([vec, zpad], axis=1), sh, axis=1)
        keep = (lane >= sh + lo) & (lane < sh + wq)
        win = out_r[row_sel, pl.ds(b128, wq + 128)]
        out_r[row_sel, pl.ds(b128, wq + 128)] = jnp.where(keep, rolled, win)

    for b in range(BB):
        at = jnp.transpose(A01[b] + l1b)
        for ib in range(N // 16):
            at_r[ib] = at[:, ib * 16:(ib + 1) * 16]
        bt = jnp.transpose(B01[b])

        for q in range(4):
            cs = 128 * q
            wq = N - cs
            btq = bt[:, cs:]
            lane = jax.lax.broadcasted_iota(jnp.int32, (2, wq + 128), 1)

            def qbody(t, carry, b=b, q=q, cs=cs, wq=wq, btq=btq, lane=lane):
                ib = q * 8 + t
                a_blk = at_r[ib]
                h = _leaky(a_blk[:, :, None] + btq[:, None, :])
                h2 = _leaky(_mm(l2w, h.reshape(EMB, 16 * wq)) + l2b)
                o = _mm(l3w, h2) + l3b
                for r in range(16):
                    i = ib * 16 + r
                    off = i * (N - 1) - (i * (i - 1)) // 2
                    store_win(slice(2 * b, 2 * b + 2), off - i + cs,
                              i + 1 - cs, o[:, r * wq:(r + 1) * wq], wq, lane)
                return carry

            jax.lax.fori_loop(0, 8, qbody, 0)


@jax.jit
def kernel(feat, bn_w, bn_b, bn2_w, bn2_b, l1_W, l1_b, l2_W, l2_b, l3_W, l3_b,
           e1a_W, e1a_b, e1b_W, e1b_b, e2a_W, e2a_b, e2b_W, e2b_b):
    feat2d = feat.reshape(BB * N, D)
    row = lambda v: v.reshape(1, -1)
    colv = lambda v: v.reshape(-1, 1)
    e1aT = e1a_W.reshape(EO, D, K).transpose(2, 1, 0).reshape(K * D, EO)
    e2aT = e2a_W.reshape(EO, EO, K).transpose(2, 1, 0).reshape(K * EO, EO)
    F = D + 2 * EO
    W1L, W1R = l1_W[:, :F], l1_W[:, F:]
    seg3 = lambda M: (M[:, :D].T, M[:, D:D + EO].T, M[:, D + EO:].T)
    segv = lambda v: (row(v[:D]), row(v[D:D + EO]), row(v[D + EO:]))
    w1l = seg3(W1L)
    w1r = seg3(W1R)
    bwl = segv(bn2_w[:F])
    bwr = segv(bn2_w[F:])
    bbl = segv(bn2_b[:F])
    bbr = segv(bn2_b[F:])

    operands = (feat2d, row(bn_w), row(bn_b),
                e1aT, row(e1a_b), e1b_W.T, row(e1b_b),
                e2aT, row(e2a_b), e2b_W.T, row(e2b_b),
                *w1l, *w1r, *bwl, *bwr, *bbl, *bbr,
                row(l1_b), l2_W, colv(l2_b), l3_W, colv(l3_b))

    out = pl.pallas_call(
        _body,
        out_shape=jax.ShapeDtypeStruct((2 * BB, 131072), jnp.float32),
        scratch_shapes=[pltpu.VMEM((N, K * N), jnp.float32),
                        pltpu.VMEM((N // 16, EMB, 16), jnp.float32)],
        interpret=_INTERP,
    )(*operands)

    preds = out[:, 1:L + 1].reshape(BB, NCLS, L).transpose(0, 2, 1)
    cells = feat[:, :, :4]
    return preds, cells
